# baseline (speedup 1.0000x reference)
import functools

import jax
import jax.numpy as jnp
from jax import lax
from jax.experimental import pallas as pl
from jax.experimental.pallas import tpu as pltpu
from jax.experimental.pallas import tpu_sc as plsc

N = 1576
D = 384
DFF = 1536
E = 6
TEMP = 0.07
EPS = 1e-6

NPAD = 2048
EPADG = 8
BN = 128
G = 30
M = 4096
DUMP = M - 1
NC = 2
NS = 16
NW = NC * NS
CH_G = M // NW
CH_T = NPAD // NW

def _mesh():
    return plsc.VectorSubcoreMesh(core_axis_name="c", subcore_axis_name="s",
                                  num_cores=NC, num_subcores=NS)


def _shift_down(a, s):
    return jnp.pad(a[:-s], ((s, 0), (0, 0)))


def _plan_body(x_ref, gw_ref, dest2_ref, gv2_ref, da_ref, db_ref, bex_ref):
    xb = x_ref[...]
    nrm = jnp.sqrt(jnp.sum(xb * xb, axis=1, keepdims=True))
    xn = xb / (nrm + EPS)
    gw = gw_ref[...]
    gn = gw / (jnp.sqrt(jnp.sum(gw * gw, axis=1, keepdims=True)) + EPS)
    logits = jnp.dot(xn, gn.T, preferred_element_type=jnp.float32) / TEMP
    cols = lax.broadcasted_iota(jnp.int32, (NPAD, EPADG), 1)
    logits = jnp.where(cols < E, logits, -1e30)
    m1 = jnp.max(logits, axis=1, keepdims=True)
    i1 = jnp.min(jnp.where(logits == m1, cols, EPADG), axis=1, keepdims=True)
    masked = jnp.where(cols == i1, -1e30, logits)
    m2 = jnp.max(masked, axis=1, keepdims=True)
    i2 = jnp.min(jnp.where(masked == m2, cols, EPADG), axis=1, keepdims=True)
    g1 = 1.0 / (1.0 + jnp.exp(m2 - m1))
    g2 = 1.0 - g1

    rows = lax.broadcasted_iota(jnp.int32, (NPAD, 1), 0)
    valid = rows < N
    oh0 = ((cols == i1) & valid).astype(jnp.int32)
    oh1 = ((cols == i2) & valid).astype(jnp.int32)
    ohs = oh0 + oh1
    s = ohs
    sh = 1
    while sh < NPAD:
        s = s + _shift_down(s, sh)
        sh *= 2
    sx = s - ohs
    cnt = jnp.sum(ohs, axis=0, keepdims=True)
    nb = (cnt + (BN - 1)) // BN
    cn = nb
    for lsh in (1, 2, 4):
        cn = cn + jnp.pad(cn[:, :-lsh], ((0, 0), (lsh, 0)))
    slotbase = (cn - nb) * BN
    rank0 = jnp.sum(oh0 * sx, axis=1, keepdims=True)
    base0 = jnp.sum(oh0.astype(jnp.float32) * slotbase.astype(jnp.float32),
                    axis=1, keepdims=True).astype(jnp.int32)
    d0 = jnp.where(valid, base0 + rank0, DUMP)
    rank1 = jnp.sum(oh1 * sx, axis=1, keepdims=True)
    base1 = jnp.sum(oh1.astype(jnp.float32) * slotbase.astype(jnp.float32),
                    axis=1, keepdims=True).astype(jnp.int32)
    d1 = jnp.where(valid, base1 + rank1, DUMP)

    dest2_ref[...] = jnp.concatenate([d0, d1], axis=1)
    validf = valid.astype(jnp.float32)
    gv2_ref[...] = jnp.concatenate([g1 * validf, g2 * validf], axis=1)
    da_ref[...] = jnp.where(valid, d0, 0)
    db_ref[...] = jnp.where(valid, d1, 0)

    grow = lax.broadcasted_iota(jnp.int32, (32, EPADG), 0)
    cmp = (jnp.broadcast_to(cn, (32, EPADG)) <= grow).astype(jnp.int32)
    bexv = jnp.clip(jnp.sum(cmp, axis=1, keepdims=True), 0, E - 1)
    bex_ref[...] = bexv


def _plan(xp, gwp):
    return pl.pallas_call(
        _plan_body,
        out_shape=[
            jax.ShapeDtypeStruct((NPAD, 2), jnp.int32),
            jax.ShapeDtypeStruct((NPAD, 2), jnp.float32),
            jax.ShapeDtypeStruct((NPAD, 1), jnp.int32),
            jax.ShapeDtypeStruct((NPAD, 1), jnp.int32),
            jax.ShapeDtypeStruct((32, 1), jnp.int32),
        ],
    )(xp, gwp)


def _build_body(dest_hbm, gv_hbm, src_hbm, gm_hbm, destv, gvv, tokv, zv,
                sem):
    wid = lax.axis_index("s") * NC + lax.axis_index("c")

    @pl.when(wid == 0)
    def _():
        pltpu.sync_copy(dest_hbm, destv)
        pltpu.sync_copy(gv_hbm, gvv)
        lane = lax.broadcasted_iota(jnp.int32, (16,), 0)
        zero16 = jnp.zeros((16,), jnp.int32)

        @pl.loop(0, M // 16)
        def _fill(i):
            off = pl.multiple_of(i * 16, 16)
            tokv[pl.ds(off, 16)] = (off + lane) >> 1
            zv[pl.ds(off, 16)] = zero16

        pltpu.sync_copy(zv, src_hbm)
        pltpu.async_copy(tokv, src_hbm.at[destv], sem).wait()
        pltpu.async_copy(gvv, gm_hbm.at[destv], sem).wait()


def _dispatch_body(src_hbm, x_hbm, xg_hbm, idxv, rowsv, sem):
    wid = lax.axis_index("s") * NC + lax.axis_index("c")
    base = pl.multiple_of(wid * CH_G, CH_G)
    pltpu.sync_copy(src_hbm.at[pl.ds(base, CH_G)], idxv)
    pltpu.async_copy(x_hbm.at[idxv], rowsv, sem).wait()
    pltpu.sync_copy(rowsv, xg_hbm.at[pl.ds(base, CH_G)])


def _ffn_body(bex_ref, xg_ref, w1_ref, b1_ref, w2_ref, b2_ref, gm_ref,
              yg_ref):
    xb16 = xg_ref[...].astype(jnp.bfloat16)
    h = jnp.dot(xb16, w1_ref[0].astype(jnp.bfloat16),
                preferred_element_type=jnp.float32)
    h = jax.nn.gelu(h.astype(jnp.bfloat16) + b1_ref[0].astype(jnp.bfloat16))
    y = jnp.dot(h, w2_ref[0].astype(jnp.bfloat16),
                preferred_element_type=jnp.float32)
    yg_ref[...] = gm_ref[...] * (y + b2_ref[0])


def _ffn(bex, xg, w1, b1, w2, b2, gm):
    grid_spec = pltpu.PrefetchScalarGridSpec(
        num_scalar_prefetch=1,
        grid=(G,),
        in_specs=[
            pl.BlockSpec((BN, D), lambda g, bex: (g, 0)),
            pl.BlockSpec((1, D, DFF), lambda g, bex: (bex[g], 0, 0)),
            pl.BlockSpec((1, 1, DFF), lambda g, bex: (bex[g], 0, 0)),
            pl.BlockSpec((1, DFF, D), lambda g, bex: (bex[g], 0, 0)),
            pl.BlockSpec((1, 1, D), lambda g, bex: (bex[g], 0, 0)),
            pl.BlockSpec((BN, 1), lambda g, bex: (g, 0)),
        ],
        out_specs=pl.BlockSpec((BN, D), lambda g, bex: (g, 0)),
    )
    return pl.pallas_call(
        _ffn_body,
        grid_spec=grid_spec,
        out_shape=jax.ShapeDtypeStruct((M, D), jnp.float32),
        compiler_params=pltpu.CompilerParams(
            dimension_semantics=("arbitrary",),
        ),
    )(bex, xg, w1, b1, w2, b2, gm)


def _combine_body(da_hbm, db_hbm, yg_hbm, out_hbm, ia, ib, ra, rb, sa, sb):
    wid = lax.axis_index("s") * NC + lax.axis_index("c")
    base = pl.multiple_of(wid * CH_T, CH_T)
    pltpu.sync_copy(da_hbm.at[pl.ds(base, CH_T)], ia)
    pltpu.sync_copy(db_hbm.at[pl.ds(base, CH_T)], ib)
    ca = pltpu.async_copy(yg_hbm.at[ia], ra, sa)
    cb = pltpu.async_copy(yg_hbm.at[ib], rb, sb)
    ca.wait()
    cb.wait()

    @pl.loop(0, CH_T)
    def _r(r):
        for c in range(D // 16):
            sl = pl.ds(c * 16, 16)
            ra[r, sl] = ra[r, sl] + rb[r, sl]

    pltpu.sync_copy(ra, out_hbm.at[pl.ds(base, CH_T)])


@jax.jit
def kernel(x, gate_w, w1, b1, w2, b2):
    xp = jnp.pad(x, ((0, NPAD - N), (0, 0)))
    gwp = jnp.pad(gate_w, ((0, EPADG - E), (0, 0)))
    dest2, gv2, da2, db2, bex2 = _plan(xp, gwp)
    build = pl.kernel(
        _build_body,
        out_type=[jax.ShapeDtypeStruct((M,), jnp.int32),
                  jax.ShapeDtypeStruct((M,), jnp.float32)],
        mesh=_mesh(),
        scratch_types=[pltpu.VMEM((M,), jnp.int32),
                       pltpu.VMEM((M,), jnp.float32),
                       pltpu.VMEM((M,), jnp.int32),
                       pltpu.VMEM((M,), jnp.int32),
                       pltpu.SemaphoreType.DMA],
    )
    dispatch = pl.kernel(
        _dispatch_body,
        out_type=jax.ShapeDtypeStruct((M, D), jnp.float32),
        mesh=_mesh(),
        scratch_types=[pltpu.VMEM((CH_G,), jnp.int32),
                       pltpu.VMEM((CH_G, D), jnp.float32),
                       pltpu.SemaphoreType.DMA],
    )
    combine = pl.kernel(
        _combine_body,
        out_type=jax.ShapeDtypeStruct((NPAD, D), jnp.float32),
        mesh=_mesh(),
        scratch_types=[pltpu.VMEM((CH_T,), jnp.int32),
                       pltpu.VMEM((CH_T,), jnp.int32),
                       pltpu.VMEM((CH_T, D), jnp.float32),
                       pltpu.VMEM((CH_T, D), jnp.float32),
                       pltpu.SemaphoreType.DMA,
                       pltpu.SemaphoreType.DMA],
    )
    src, gm = build(dest2.reshape(M), gv2.reshape(M))
    xg = dispatch(src, xp)
    yg = _ffn(bex2.reshape(32), xg, w1, b1[:, None, :], w2, b2[:, None, :],
              gm[:, None])
    out = combine(da2.reshape(NPAD), db2.reshape(NPAD), yg)
    return out[:N]


# speedup vs baseline: 1.8805x; 1.8805x over previous
import functools

import jax
import jax.numpy as jnp
from jax import lax
from jax.experimental import pallas as pl
from jax.experimental.pallas import tpu as pltpu
from jax.experimental.pallas import tpu_sc as plsc

N = 1576
D = 384
DFF = 1536
E = 6
TEMP = 0.07
EPS = 1e-6

NPAD = 2048
EPADG = 8
BN = 128
G = 30
M = 4096
DUMP = M - 1
NC = 2
NS = 16
NW = NC * NS
CH_G = M // NW
CH_T = NPAD // NW

def _mesh():
    return plsc.VectorSubcoreMesh(core_axis_name="c", subcore_axis_name="s",
                                  num_cores=NC, num_subcores=NS)


def _shift_down(a, s):
    return jnp.pad(a[:-s], ((s, 0), (0, 0)))


def _plan_body(x_ref, gw_ref, dest2_ref, gv2_ref, da_ref, db_ref, bex_ref):
    xb = x_ref[...]
    nrm = jnp.sqrt(jnp.sum(xb * xb, axis=1, keepdims=True))
    xn = xb / (nrm + EPS)
    gw = gw_ref[...]
    gn = gw / (jnp.sqrt(jnp.sum(gw * gw, axis=1, keepdims=True)) + EPS)
    logits = jnp.dot(xn, gn.T, preferred_element_type=jnp.float32) / TEMP
    cols = lax.broadcasted_iota(jnp.int32, (NPAD, EPADG), 1)
    logits = jnp.where(cols < E, logits, -1e30)
    m1 = jnp.max(logits, axis=1, keepdims=True)
    i1 = jnp.min(jnp.where(logits == m1, cols, EPADG), axis=1, keepdims=True)
    masked = jnp.where(cols == i1, -1e30, logits)
    m2 = jnp.max(masked, axis=1, keepdims=True)
    i2 = jnp.min(jnp.where(masked == m2, cols, EPADG), axis=1, keepdims=True)
    g1 = 1.0 / (1.0 + jnp.exp(m2 - m1))
    g2 = 1.0 - g1

    rows = lax.broadcasted_iota(jnp.int32, (NPAD, 1), 0)
    valid = rows < N
    oh0 = ((cols == i1) & valid).astype(jnp.int32)
    oh1 = ((cols == i2) & valid).astype(jnp.int32)
    ohs = oh0 + oh1
    s = ohs
    sh = 1
    while sh < NPAD:
        s = s + _shift_down(s, sh)
        sh *= 2
    sx = s - ohs
    cnt = jnp.sum(ohs, axis=0, keepdims=True)
    nb = (cnt + (BN - 1)) // BN
    cn = nb
    for lsh in (1, 2, 4):
        cn = cn + jnp.pad(cn[:, :-lsh], ((0, 0), (lsh, 0)))
    slotbase = (cn - nb) * BN
    rank0 = jnp.sum(oh0 * sx, axis=1, keepdims=True)
    base0 = jnp.sum(oh0.astype(jnp.float32) * slotbase.astype(jnp.float32),
                    axis=1, keepdims=True).astype(jnp.int32)
    d0 = jnp.where(valid, base0 + rank0, DUMP)
    rank1 = jnp.sum(oh1 * sx, axis=1, keepdims=True)
    base1 = jnp.sum(oh1.astype(jnp.float32) * slotbase.astype(jnp.float32),
                    axis=1, keepdims=True).astype(jnp.int32)
    d1 = jnp.where(valid, base1 + rank1, DUMP)

    dest2_ref[...] = jnp.concatenate([d0, d1], axis=1)
    validf = valid.astype(jnp.float32)
    gv2_ref[...] = jnp.concatenate([g1 * validf, g2 * validf], axis=1)
    da_ref[...] = jnp.where(valid, d0, 0)
    db_ref[...] = jnp.where(valid, d1, 0)

    grow = lax.broadcasted_iota(jnp.int32, (32, EPADG), 0)
    cmp = (jnp.broadcast_to(cn, (32, EPADG)) <= grow).astype(jnp.int32)
    bexv = jnp.clip(jnp.sum(cmp, axis=1, keepdims=True), 0, E - 1)
    bex_ref[...] = bexv


def _plan(xp, gwp):
    return pl.pallas_call(
        _plan_body,
        out_shape=[
            jax.ShapeDtypeStruct((NPAD, 2), jnp.int32),
            jax.ShapeDtypeStruct((NPAD, 2), jnp.float32),
            jax.ShapeDtypeStruct((NPAD, 1), jnp.int32),
            jax.ShapeDtypeStruct((NPAD, 1), jnp.int32),
            jax.ShapeDtypeStruct((32, 1), jnp.int32),
        ],
    )(xp, gwp)


BM = 256


def _invert_body(dest_ref, gv_ref, src_ref, gm_ref):
    g = pl.program_id(0)
    mrow = g * BM + lax.broadcasted_iota(jnp.int32, (BM, 1), 0)
    z = (dest_ref[...] == mrow).astype(jnp.float32)
    acol = lax.broadcasted_iota(jnp.int32, (M, 1), 0)
    tokcol = (acol >> 1).astype(jnp.float32)
    v = jnp.concatenate([tokcol, gv_ref[...]], axis=1)
    r = jnp.dot(z, v, preferred_element_type=jnp.float32,
                precision=jax.lax.Precision.HIGHEST)
    src_ref[...] = jnp.clip(r[:, :1], 0, NPAD - 1).astype(jnp.int32)
    gm_ref[...] = r[:, 1:2]


def _invert(dest_row, gv_col):
    return pl.pallas_call(
        _invert_body,
        grid=(M // BM,),
        in_specs=[
            pl.BlockSpec((1, M), lambda g: (0, 0)),
            pl.BlockSpec((M, 1), lambda g: (0, 0)),
        ],
        out_specs=[
            pl.BlockSpec((BM, 1), lambda g: (g, 0)),
            pl.BlockSpec((BM, 1), lambda g: (g, 0)),
        ],
        out_shape=[
            jax.ShapeDtypeStruct((M, 1), jnp.int32),
            jax.ShapeDtypeStruct((M, 1), jnp.float32),
        ],
        compiler_params=pltpu.CompilerParams(
            dimension_semantics=("arbitrary",),
        ),
    )(dest_row, gv_col)


def _dispatch_body(src_hbm, x_hbm, xg_hbm, idxv, rowsv, sem):
    wid = lax.axis_index("s") * NC + lax.axis_index("c")
    base = pl.multiple_of(wid * CH_G, CH_G)
    pltpu.sync_copy(src_hbm.at[pl.ds(base, CH_G)], idxv)
    pltpu.async_copy(x_hbm.at[idxv], rowsv, sem).wait()
    pltpu.sync_copy(rowsv, xg_hbm.at[pl.ds(base, CH_G)])


def _ffn_body(bex_ref, xg_ref, w1_ref, b1_ref, w2_ref, b2_ref, gm_ref,
              yg_ref):
    xb16 = xg_ref[...].astype(jnp.bfloat16)
    h = jnp.dot(xb16, w1_ref[0].astype(jnp.bfloat16),
                preferred_element_type=jnp.float32)
    h = jax.nn.gelu(h.astype(jnp.bfloat16) + b1_ref[0].astype(jnp.bfloat16))
    y = jnp.dot(h, w2_ref[0].astype(jnp.bfloat16),
                preferred_element_type=jnp.float32)
    yg_ref[...] = gm_ref[...] * (y + b2_ref[0])


def _ffn(bex, xg, w1, b1, w2, b2, gm):
    grid_spec = pltpu.PrefetchScalarGridSpec(
        num_scalar_prefetch=1,
        grid=(G,),
        in_specs=[
            pl.BlockSpec((BN, D), lambda g, bex: (g, 0)),
            pl.BlockSpec((1, D, DFF), lambda g, bex: (bex[g], 0, 0)),
            pl.BlockSpec((1, 1, DFF), lambda g, bex: (bex[g], 0, 0)),
            pl.BlockSpec((1, DFF, D), lambda g, bex: (bex[g], 0, 0)),
            pl.BlockSpec((1, 1, D), lambda g, bex: (bex[g], 0, 0)),
            pl.BlockSpec((BN, 1), lambda g, bex: (g, 0)),
        ],
        out_specs=pl.BlockSpec((BN, D), lambda g, bex: (g, 0)),
    )
    return pl.pallas_call(
        _ffn_body,
        grid_spec=grid_spec,
        out_shape=jax.ShapeDtypeStruct((M, D), jnp.float32),
        compiler_params=pltpu.CompilerParams(
            dimension_semantics=("arbitrary",),
        ),
    )(bex, xg, w1, b1, w2, b2, gm)


def _combine_body(da_hbm, db_hbm, yg_hbm, out_hbm, ia, ib, ra, rb, sa, sb):
    wid = lax.axis_index("s") * NC + lax.axis_index("c")
    base = pl.multiple_of(wid * CH_T, CH_T)
    pltpu.sync_copy(da_hbm.at[pl.ds(base, CH_T)], ia)
    pltpu.sync_copy(db_hbm.at[pl.ds(base, CH_T)], ib)
    ca = pltpu.async_copy(yg_hbm.at[ia], ra, sa)
    cb = pltpu.async_copy(yg_hbm.at[ib], rb, sb)
    ca.wait()
    cb.wait()

    @pl.loop(0, CH_T)
    def _r(r):
        for c in range(D // 16):
            sl = pl.ds(c * 16, 16)
            ra[r, sl] = ra[r, sl] + rb[r, sl]

    pltpu.sync_copy(ra, out_hbm.at[pl.ds(base, CH_T)])


@jax.jit
def kernel(x, gate_w, w1, b1, w2, b2):
    xp = jnp.pad(x, ((0, NPAD - N), (0, 0)))
    gwp = jnp.pad(gate_w, ((0, EPADG - E), (0, 0)))
    dest2, gv2, da2, db2, bex2 = _plan(xp, gwp)
    dispatch = pl.kernel(
        _dispatch_body,
        out_type=jax.ShapeDtypeStruct((M, D), jnp.float32),
        mesh=_mesh(),
        scratch_types=[pltpu.VMEM((CH_G,), jnp.int32),
                       pltpu.VMEM((CH_G, D), jnp.float32),
                       pltpu.SemaphoreType.DMA],
    )
    combine = pl.kernel(
        _combine_body,
        out_type=jax.ShapeDtypeStruct((NPAD, D), jnp.float32),
        mesh=_mesh(),
        scratch_types=[pltpu.VMEM((CH_T,), jnp.int32),
                       pltpu.VMEM((CH_T,), jnp.int32),
                       pltpu.VMEM((CH_T, D), jnp.float32),
                       pltpu.VMEM((CH_T, D), jnp.float32),
                       pltpu.SemaphoreType.DMA,
                       pltpu.SemaphoreType.DMA],
    )
    src, gm = _invert(dest2.reshape(1, M), gv2.reshape(M, 1))
    xg = dispatch(src.reshape(M), xp)
    yg = _ffn(bex2.reshape(32), xg, w1, b1[:, None, :], w2, b2[:, None, :],
              gm)
    out = combine(da2.reshape(NPAD), db2.reshape(NPAD), yg)
    return out[:N]


# speedup vs baseline: 3.5434x; 1.8843x over previous
import jax
import jax.numpy as jnp
from jax import lax
from jax.experimental import pallas as pl
from jax.experimental.pallas import tpu as pltpu

N = 1576
D = 384
DFF = 1536
E = 6
TEMP = 0.07
EPS = 1e-6

NPAD = 2048
EPADG = 8
BN = 128
G = 30
MS = G * BN
DUMP = NPAD * 4
BT = 256


def _shift_down(a, s):
    return jnp.pad(a[:-s], ((s, 0), (0, 0)))


def _plan_body(x_ref, gw_ref, da_ref, db_ref, ga_ref, gb_ref, bex_ref,
               x16_ref):
    xb = x_ref[...]
    nrm = jnp.sqrt(jnp.sum(xb * xb, axis=1, keepdims=True))
    xn = xb / (nrm + EPS)
    gw = gw_ref[...]
    gn = gw / (jnp.sqrt(jnp.sum(gw * gw, axis=1, keepdims=True)) + EPS)
    logits = jnp.dot(xn, gn.T, preferred_element_type=jnp.float32) / TEMP
    cols = lax.broadcasted_iota(jnp.int32, (NPAD, EPADG), 1)
    logits = jnp.where(cols < E, logits, -1e30)
    m1 = jnp.max(logits, axis=1, keepdims=True)
    i1 = jnp.min(jnp.where(logits == m1, cols, EPADG), axis=1, keepdims=True)
    masked = jnp.where(cols == i1, -1e30, logits)
    m2 = jnp.max(masked, axis=1, keepdims=True)
    i2 = jnp.min(jnp.where(masked == m2, cols, EPADG), axis=1, keepdims=True)
    g1 = 1.0 / (1.0 + jnp.exp(m2 - m1))
    g2 = 1.0 - g1

    rows = lax.broadcasted_iota(jnp.int32, (NPAD, 1), 0)
    valid = rows < N
    oh0 = ((cols == i1) & valid).astype(jnp.int32)
    oh1 = ((cols == i2) & valid).astype(jnp.int32)
    ohs = oh0 + oh1
    s = ohs
    sh = 1
    while sh < NPAD:
        s = s + _shift_down(s, sh)
        sh *= 2
    sx = s - ohs
    cnt = jnp.sum(ohs, axis=0, keepdims=True)
    nb = (cnt + (BN - 1)) // BN
    cn = nb
    for lsh in (1, 2, 4):
        cn = cn + jnp.pad(cn[:, :-lsh], ((0, 0), (lsh, 0)))
    slotbase = (cn - nb) * BN
    rank0 = jnp.sum(oh0 * sx, axis=1, keepdims=True)
    base0 = jnp.sum(oh0 * jnp.broadcast_to(slotbase, (NPAD, EPADG)),
                    axis=1, keepdims=True)
    d0 = jnp.where(valid, base0 + rank0, DUMP)
    rank1 = jnp.sum(oh1 * sx, axis=1, keepdims=True)
    base1 = jnp.sum(oh1 * jnp.broadcast_to(slotbase, (NPAD, EPADG)),
                    axis=1, keepdims=True)
    d1 = jnp.where(valid, base1 + rank1, DUMP)

    da_ref[...] = d0
    db_ref[...] = d1
    validf = valid.astype(jnp.float32)
    ga_ref[...] = g1 * validf
    gb_ref[...] = g2 * validf

    grow = lax.broadcasted_iota(jnp.int32, (32, EPADG), 0)
    cmp = (jnp.broadcast_to(cn, (32, EPADG)) <= grow).astype(jnp.int32)
    bex_ref[...] = jnp.clip(jnp.sum(cmp, axis=1, keepdims=True), 0, E - 1)

    x16_ref[...] = xb.astype(jnp.bfloat16)


def _plan(xp, gwp):
    return pl.pallas_call(
        _plan_body,
        out_shape=[
            jax.ShapeDtypeStruct((NPAD, 1), jnp.int32),
            jax.ShapeDtypeStruct((NPAD, 1), jnp.int32),
            jax.ShapeDtypeStruct((NPAD, 1), jnp.float32),
            jax.ShapeDtypeStruct((NPAD, 1), jnp.float32),
            jax.ShapeDtypeStruct((32, 1), jnp.int32),
            jax.ShapeDtypeStruct((NPAD, D), jnp.bfloat16),
        ],
    )(xp, gwp)


def _ffn_body(bex_ref, da_ref, db_ref, ga_ref, gb_ref, x16_ref,
              w1_ref, b1_ref, w2_ref, b2_ref, yg_ref):
    g = pl.program_id(0)
    mrow = g * BN + lax.broadcasted_iota(jnp.int32, (BN, 1), 0)
    za = da_ref[...] == mrow
    zb = db_ref[...] == mrow
    gm = (jnp.dot(za.astype(jnp.float32), ga_ref[...],
                  preferred_element_type=jnp.float32,
                  precision=jax.lax.Precision.HIGHEST)
          + jnp.dot(zb.astype(jnp.float32), gb_ref[...],
                    preferred_element_type=jnp.float32,
                    precision=jax.lax.Precision.HIGHEST))
    z16 = (za | zb).astype(jnp.bfloat16)
    xg = jnp.dot(z16, x16_ref[...],
                 preferred_element_type=jnp.float32)
    h = jnp.dot(xg.astype(jnp.bfloat16), w1_ref[0].astype(jnp.bfloat16),
                preferred_element_type=jnp.float32)
    h = jax.nn.gelu(h.astype(jnp.bfloat16) + b1_ref[0].astype(jnp.bfloat16))
    y = jnp.dot(h, w2_ref[0].astype(jnp.bfloat16),
                preferred_element_type=jnp.float32)
    yg_ref[...] = (gm * (y + b2_ref[0])).astype(jnp.bfloat16)


def _ffn(bex, da_row, db_row, ga_col, gb_col, x16, w1, b1, w2, b2):
    grid_spec = pltpu.PrefetchScalarGridSpec(
        num_scalar_prefetch=1,
        grid=(G,),
        in_specs=[
            pl.BlockSpec((1, NPAD), lambda g, bex: (0, 0)),
            pl.BlockSpec((1, NPAD), lambda g, bex: (0, 0)),
            pl.BlockSpec((NPAD, 1), lambda g, bex: (0, 0)),
            pl.BlockSpec((NPAD, 1), lambda g, bex: (0, 0)),
            pl.BlockSpec((NPAD, D), lambda g, bex: (0, 0)),
            pl.BlockSpec((1, D, DFF), lambda g, bex: (bex[g], 0, 0)),
            pl.BlockSpec((1, 1, DFF), lambda g, bex: (bex[g], 0, 0)),
            pl.BlockSpec((1, DFF, D), lambda g, bex: (bex[g], 0, 0)),
            pl.BlockSpec((1, 1, D), lambda g, bex: (bex[g], 0, 0)),
        ],
        out_specs=pl.BlockSpec((BN, D), lambda g, bex: (g, 0)),
    )
    return pl.pallas_call(
        _ffn_body,
        grid_spec=grid_spec,
        out_shape=jax.ShapeDtypeStruct((MS, D), jnp.bfloat16),
        compiler_params=pltpu.CompilerParams(
            dimension_semantics=("arbitrary",),
        ),
    )(bex, da_row, db_row, ga_col, gb_col, x16, w1, b1, w2, b2)


def _combine_body(da_ref, db_ref, yg_ref, out_ref):
    mlane = lax.broadcasted_iota(jnp.int32, (BT, MS), 1)
    a = (da_ref[...] == mlane) | (db_ref[...] == mlane)
    out_ref[...] = jnp.dot(a.astype(jnp.bfloat16), yg_ref[...],
                           preferred_element_type=jnp.float32)


def _combine(da_col, db_col, yg):
    return pl.pallas_call(
        _combine_body,
        grid=(NPAD // BT,),
        in_specs=[
            pl.BlockSpec((BT, 1), lambda g: (g, 0)),
            pl.BlockSpec((BT, 1), lambda g: (g, 0)),
            pl.BlockSpec((MS, D), lambda g: (0, 0)),
        ],
        out_specs=pl.BlockSpec((BT, D), lambda g: (g, 0)),
        out_shape=jax.ShapeDtypeStruct((NPAD, D), jnp.float32),
        compiler_params=pltpu.CompilerParams(
            dimension_semantics=("arbitrary",),
        ),
    )(da_col, db_col, yg)


@jax.jit
def kernel(x, gate_w, w1, b1, w2, b2):
    xp = jnp.pad(x, ((0, NPAD - N), (0, 0)))
    gwp = jnp.pad(gate_w, ((0, EPADG - E), (0, 0)))
    da, db, ga, gb, bex, x16 = _plan(xp, gwp)
    yg = _ffn(bex.reshape(32), da.reshape(1, NPAD), db.reshape(1, NPAD),
              ga, gb, x16, w1, b1[:, None, :], w2, b2[:, None, :])
    out = _combine(da, db, yg)
    return out[:N]


# speedup vs baseline: 5.0220x; 1.4173x over previous
import jax
import jax.numpy as jnp
from jax import lax
from jax.experimental import pallas as pl
from jax.experimental.pallas import tpu as pltpu

N = 1576
D = 384
DFF = 1536
E = 6
TEMP = 0.07
EPS = 1e-6

NPAD = 2048
EPADG = 8
BN = 128
G = 30
MS = G * BN
DUMP = NPAD * 4
BT = 256


def _shift_down(a, s):
    return jnp.pad(a[:-s], ((s, 0), (0, 0)))


def _plan_body(x_ref, gw_ref, da_ref, db_ref, ga_ref, gb_ref, bex_ref,
               x16_ref):
    xb = x_ref[...]
    nrm = jnp.sqrt(jnp.sum(xb * xb, axis=1, keepdims=True))
    xn = xb / (nrm + EPS)
    gw = gw_ref[...]
    gn = gw / (jnp.sqrt(jnp.sum(gw * gw, axis=1, keepdims=True)) + EPS)
    logits = jnp.dot(xn, gn.T, preferred_element_type=jnp.float32) / TEMP
    cols = lax.broadcasted_iota(jnp.int32, (NPAD, EPADG), 1)
    logits = jnp.where(cols < E, logits, -1e30)
    m1 = jnp.max(logits, axis=1, keepdims=True)
    i1 = jnp.min(jnp.where(logits == m1, cols, EPADG), axis=1, keepdims=True)
    masked = jnp.where(cols == i1, -1e30, logits)
    m2 = jnp.max(masked, axis=1, keepdims=True)
    i2 = jnp.min(jnp.where(masked == m2, cols, EPADG), axis=1, keepdims=True)
    g1 = 1.0 / (1.0 + jnp.exp(m2 - m1))
    g2 = 1.0 - g1

    rows = lax.broadcasted_iota(jnp.int32, (NPAD, 1), 0)
    valid = rows < N
    oh0 = ((cols == i1) & valid).astype(jnp.int32)
    oh1 = ((cols == i2) & valid).astype(jnp.int32)
    ohs = oh0 + oh1
    s = ohs
    sh = 1
    while sh < NPAD:
        s = s + _shift_down(s, sh)
        sh *= 2
    sx = s - ohs
    cnt = jnp.sum(ohs, axis=0, keepdims=True)
    nb = (cnt + (BN - 1)) // BN
    cn = nb
    for lsh in (1, 2, 4):
        cn = cn + jnp.pad(cn[:, :-lsh], ((0, 0), (lsh, 0)))
    slotbase = (cn - nb) * BN
    rank0 = jnp.sum(oh0 * sx, axis=1, keepdims=True)
    base0 = jnp.sum(oh0 * jnp.broadcast_to(slotbase, (NPAD, EPADG)),
                    axis=1, keepdims=True)
    d0 = jnp.where(valid, base0 + rank0, DUMP)
    rank1 = jnp.sum(oh1 * sx, axis=1, keepdims=True)
    base1 = jnp.sum(oh1 * jnp.broadcast_to(slotbase, (NPAD, EPADG)),
                    axis=1, keepdims=True)
    d1 = jnp.where(valid, base1 + rank1, DUMP)

    da_ref[...] = d0
    db_ref[...] = d1
    validf = valid.astype(jnp.float32)
    ga_ref[...] = g1 * validf
    gb_ref[...] = g2 * validf

    grow = lax.broadcasted_iota(jnp.int32, (32, EPADG), 0)
    cmp = (jnp.broadcast_to(cn, (32, EPADG)) <= grow).astype(jnp.int32)
    bex_ref[...] = jnp.clip(jnp.sum(cmp, axis=1, keepdims=True), 0, E - 1)

    x16_ref[...] = xb.astype(jnp.bfloat16)


def _plan(xp, gwp):
    return pl.pallas_call(
        _plan_body,
        out_shape=[
            jax.ShapeDtypeStruct((NPAD, 1), jnp.int32),
            jax.ShapeDtypeStruct((NPAD, 1), jnp.int32),
            jax.ShapeDtypeStruct((NPAD, 1), jnp.float32),
            jax.ShapeDtypeStruct((NPAD, 1), jnp.float32),
            jax.ShapeDtypeStruct((32, 1), jnp.int32),
            jax.ShapeDtypeStruct((NPAD, D), jnp.bfloat16),
        ],
    )(xp, gwp)


def _ffn_body(bex_ref, da_ref, db_ref, ga_ref, gb_ref, x16_ref,
              w1_ref, b1_ref, w2_ref, b2_ref, yg_ref, w1c_ref, w2c_ref):
    g = pl.program_id(0)
    mrow = g * BN + lax.broadcasted_iota(jnp.int32, (BN, 1), 0)
    za = da_ref[...] == mrow
    zb = db_ref[...] == mrow
    gm = (jnp.sum(jnp.where(za, ga_ref[...], 0.0), axis=1, keepdims=True)
          + jnp.sum(jnp.where(zb, gb_ref[...], 0.0), axis=1, keepdims=True))
    z16 = (za | zb).astype(jnp.bfloat16)
    xg = jnp.dot(z16, x16_ref[...],
                 preferred_element_type=jnp.float32)

    new_expert = jnp.logical_or(
        g == 0, bex_ref[g] != bex_ref[jnp.maximum(g - 1, 0)])

    @pl.when(new_expert)
    def _cache():
        w1c_ref[...] = w1_ref[0].astype(jnp.bfloat16)
        w2c_ref[...] = w2_ref[0].astype(jnp.bfloat16)

    h = jnp.dot(xg.astype(jnp.bfloat16), w1c_ref[...],
                preferred_element_type=jnp.float32)
    h = jax.nn.gelu(h.astype(jnp.bfloat16) + b1_ref[0].astype(jnp.bfloat16))
    y = jnp.dot(h, w2c_ref[...],
                preferred_element_type=jnp.float32)
    yg_ref[...] = (gm * (y + b2_ref[0])).astype(jnp.bfloat16)


def _ffn(bex, da_row, db_row, ga_col, gb_col, x16, w1, b1, w2, b2):
    grid_spec = pltpu.PrefetchScalarGridSpec(
        num_scalar_prefetch=1,
        grid=(G,),
        in_specs=[
            pl.BlockSpec((1, NPAD), lambda g, bex: (0, 0)),
            pl.BlockSpec((1, NPAD), lambda g, bex: (0, 0)),
            pl.BlockSpec((1, NPAD), lambda g, bex: (0, 0)),
            pl.BlockSpec((1, NPAD), lambda g, bex: (0, 0)),
            pl.BlockSpec((NPAD, D), lambda g, bex: (0, 0)),
            pl.BlockSpec((1, D, DFF), lambda g, bex: (bex[g], 0, 0)),
            pl.BlockSpec((1, 1, DFF), lambda g, bex: (bex[g], 0, 0)),
            pl.BlockSpec((1, DFF, D), lambda g, bex: (bex[g], 0, 0)),
            pl.BlockSpec((1, 1, D), lambda g, bex: (bex[g], 0, 0)),
        ],
        out_specs=pl.BlockSpec((BN, D), lambda g, bex: (g, 0)),
        scratch_shapes=[pltpu.VMEM((D, DFF), jnp.bfloat16),
                        pltpu.VMEM((DFF, D), jnp.bfloat16)],
    )
    return pl.pallas_call(
        _ffn_body,
        grid_spec=grid_spec,
        out_shape=jax.ShapeDtypeStruct((MS, D), jnp.bfloat16),
        compiler_params=pltpu.CompilerParams(
            dimension_semantics=("arbitrary",),
        ),
    )(bex, da_row, db_row, ga_col, gb_col, x16, w1, b1, w2, b2)


def _combine_body(da_ref, db_ref, yg_ref, out_ref):
    mlane = lax.broadcasted_iota(jnp.int32, (BT, MS), 1)
    a = (da_ref[...] == mlane) | (db_ref[...] == mlane)
    out_ref[...] = jnp.dot(a.astype(jnp.bfloat16), yg_ref[...],
                           preferred_element_type=jnp.float32)


def _combine(da_col, db_col, yg):
    return pl.pallas_call(
        _combine_body,
        grid=(NPAD // BT,),
        in_specs=[
            pl.BlockSpec((BT, 1), lambda g: (g, 0)),
            pl.BlockSpec((BT, 1), lambda g: (g, 0)),
            pl.BlockSpec((MS, D), lambda g: (0, 0)),
        ],
        out_specs=pl.BlockSpec((BT, D), lambda g: (g, 0)),
        out_shape=jax.ShapeDtypeStruct((NPAD, D), jnp.float32),
        compiler_params=pltpu.CompilerParams(
            dimension_semantics=("arbitrary",),
        ),
    )(da_col, db_col, yg)


@jax.jit
def kernel(x, gate_w, w1, b1, w2, b2):
    xp = jnp.pad(x, ((0, NPAD - N), (0, 0)))
    gwp = jnp.pad(gate_w, ((0, EPADG - E), (0, 0)))
    da, db, ga, gb, bex, x16 = _plan(xp, gwp)
    yg = _ffn(bex.reshape(32), da.reshape(1, NPAD), db.reshape(1, NPAD),
              ga.reshape(1, NPAD), gb.reshape(1, NPAD), x16,
              w1, b1[:, None, :], w2, b2[:, None, :])
    out = _combine(da, db, yg)
    return out[:N]


# speedup vs baseline: 8.4110x; 1.6748x over previous
import jax
import jax.numpy as jnp
from jax.experimental import pallas as pl
from jax.experimental.pallas import tpu as pltpu

N = 1576
D = 384
DFF = 1536
E = 6
TEMP = 0.07
EPS = 1e-6

NPAD = 1600
EPADG = 8


def _moe_body(x_ref, gwn_ref, w1_ref, b1_ref, w2_ref, b2_ref, out_ref,
              comb_ref):
    j = pl.program_id(0)

    @pl.when(j == 0)
    def _router():
        xb = x_ref[...]
        nrm = jnp.sqrt(jnp.sum(xb * xb, axis=1, keepdims=True))
        xn = xb / (nrm + EPS)
        gw = gwn_ref[...]
        gn = gw / (jnp.sqrt(jnp.sum(gw * gw, axis=1, keepdims=True)) + EPS)
        logits = jnp.dot(xn, gn.T,
                         preferred_element_type=jnp.float32) / TEMP
        cols = jax.lax.broadcasted_iota(jnp.int32, (NPAD, EPADG), 1)
        logits = jnp.where(cols < E, logits, -1e30)
        m1 = jnp.max(logits, axis=1, keepdims=True)
        i1 = jnp.min(jnp.where(logits == m1, cols, EPADG), axis=1,
                     keepdims=True)
        masked = jnp.where(cols == i1, -1e30, logits)
        m2 = jnp.max(masked, axis=1, keepdims=True)
        i2 = jnp.min(jnp.where(masked == m2, cols, EPADG), axis=1,
                     keepdims=True)
        g1 = 1.0 / (1.0 + jnp.exp(m2 - m1))
        g2 = 1.0 - g1
        comb_ref[...] = (g1 * (cols == i1).astype(jnp.float32)
                         + g2 * (cols == i2).astype(jnp.float32))

    xb16 = x_ref[...].astype(jnp.bfloat16)
    h = jnp.dot(xb16, w1_ref[0].astype(jnp.bfloat16),
                preferred_element_type=jnp.float32)
    h = jax.nn.gelu(h.astype(jnp.bfloat16)
                    + b1_ref[0].astype(jnp.bfloat16))
    y = jnp.dot(h, w2_ref[0].astype(jnp.bfloat16),
                preferred_element_type=jnp.float32)
    y = y + b2_ref[0]
    allcols = jax.lax.broadcasted_iota(jnp.int32, (NPAD, EPADG), 1)
    cb = jnp.sum(jnp.where(allcols == j, comb_ref[...], 0.0), axis=1,
                 keepdims=True)
    contrib = cb * y

    @pl.when(j == 0)
    def _init():
        out_ref[...] = contrib

    @pl.when(j > 0)
    def _acc():
        out_ref[...] += contrib


@jax.jit
def kernel(x, gate_w, w1, b1, w2, b2):
    xp = jnp.pad(x, ((0, NPAD - N), (0, 0)))
    gwp = jnp.pad(gate_w, ((0, EPADG - E), (0, 0)))

    out = pl.pallas_call(
        _moe_body,
        grid=(E,),
        in_specs=[
            pl.BlockSpec((NPAD, D), lambda j: (0, 0)),
            pl.BlockSpec((EPADG, D), lambda j: (0, 0)),
            pl.BlockSpec((1, D, DFF), lambda j: (j, 0, 0)),
            pl.BlockSpec((1, 1, DFF), lambda j: (j, 0, 0)),
            pl.BlockSpec((1, DFF, D), lambda j: (j, 0, 0)),
            pl.BlockSpec((1, 1, D), lambda j: (j, 0, 0)),
        ],
        out_specs=pl.BlockSpec((NPAD, D), lambda j: (0, 0)),
        out_shape=jax.ShapeDtypeStruct((NPAD, D), jnp.float32),
        scratch_shapes=[pltpu.VMEM((NPAD, EPADG), jnp.float32)],
        compiler_params=pltpu.CompilerParams(
            dimension_semantics=("arbitrary",),
        ),
    )(xp, gwp, w1, b1[:, None, :], w2, b2[:, None, :])
    return out[:N]
